# baseline (speedup 1.0000x reference)
import functools

import jax
import jax.numpy as jnp
from jax import lax
from jax.experimental import pallas as pl
from jax.experimental.pallas import tpu as pltpu
from jax.experimental.pallas import tpu_sc as plsc

_B = 128
_V = 100000
_V_TC = 81920
_V_SC = _V - _V_TC
_ROWS = 8
_CW = 20480
_NCB = _V_TC // _CW
_LANES = 128

_NW = 32
_SC_ROWS = _B // _NW
_SC_VREGS = _V_SC // 16
_SC_UNROLL = 8
_SC_TRIPS = _SC_VREGS // _SC_UNROLL
_SC_TAIL = _SC_VREGS - _SC_TRIPS * _SC_UNROLL

_K0 = 0
_K1 = 42
_K2 = 0x1BD11BDA ^ _K0 ^ _K1
_ROT_A = (13, 15, 26, 6)
_ROT_B = (17, 29, 16, 24)

_LN2 = 0.69314718055994530942


def _threefry_bits(x1):
    u32 = lambda v: jnp.uint32(v)

    def four_rounds(x0, x1, rots):
        for d in rots:
            x0 = x0 + x1
            x1 = (x1 << u32(d)) | (x1 >> u32(32 - d))
            x1 = x1 ^ x0
        return x0, x1

    x0 = x1
    x1 = (x1 << u32(_ROT_A[0])) | (x1 >> u32(32 - _ROT_A[0]))
    x1 = x1 ^ x0
    for d in _ROT_A[1:]:
        x0 = x0 + x1
        x1 = (x1 << u32(d)) | (x1 >> u32(32 - d))
        x1 = x1 ^ x0

    x0 = x0 + u32(_K1); x1 = x1 + u32(_K2 + 1)
    x0, x1 = four_rounds(x0, x1, _ROT_B)
    x0 = x0 + u32(_K2); x1 = x1 + u32(_K0 + 2)
    x0, x1 = four_rounds(x0, x1, _ROT_A)
    x0 = x0 + u32(_K0); x1 = x1 + u32(_K1 + 3)
    x0, x1 = four_rounds(x0, x1, _ROT_B)
    x0 = x0 + u32(_K1); x1 = x1 + u32(_K2 + 4)
    x0, x1 = four_rounds(x0, x1, _ROT_A)
    x0 = x0 + u32(_K2); x1 = x1 + u32(_K0 + 5)
    return x0 ^ x1


def _bits_to_noise(bits):
    fb = lax.bitcast_convert_type(
        (bits >> jnp.uint32(9)) | jnp.uint32(0x3F800000), jnp.float32)
    uni = fb - jnp.float32(1.0)
    expo = -jnp.log1p(-uni)
    return jnp.log(expo + jnp.float32(1e-10))


def _pass1_body(logits_ref, temp_ref, val_ref, idx_ref):
    r = pl.program_id(0)
    c = pl.program_id(1)

    temp = jnp.broadcast_to(temp_ref[...], (_ROWS, _LANES))
    t0 = temp == jnp.float32(0.0)
    neg_inf = jnp.float32(-jnp.inf)

    lane = lax.broadcasted_iota(jnp.int32, (_ROWS, _LANES), 1)
    base_col = lane + c * _CW
    row = lax.broadcasted_iota(jnp.int32, (_ROWS, _LANES), 0) + r * _ROWS
    cnt_base = (row * _V + base_col + jnp.int32(_K1)).astype(jnp.uint32)

    first = c == 0
    acc_v = jnp.where(first, jnp.full((_ROWS, _LANES), neg_inf), val_ref[...])
    acc_i = jnp.where(first, jnp.zeros((_ROWS, _LANES), jnp.int32), idx_ref[...])

    for k in range(_CW // _LANES):
        col = base_col + jnp.int32(k * _LANES)
        bits = _threefry_bits(cnt_base + jnp.uint32(k * _LANES))
        noise = _bits_to_noise(bits)

        lg = logits_ref[:, k * _LANES:(k + 1) * _LANES]
        s = lg / temp - noise
        se = jnp.where(t0, lg, s)

        upd = se > acc_v
        acc_v = jnp.where(upd, se, acc_v)
        acc_i = jnp.where(upd, col, acc_i)

    val_ref[...] = acc_v
    idx_ref[...] = acc_i


def _sc_ln(x):
    xb = lax.bitcast_convert_type(x, jnp.uint32)
    e = (xb >> jnp.uint32(23)).astype(jnp.int32) - 127
    mb = (xb & jnp.uint32(0x7FFFFF)) | jnp.uint32(0x3F800000)
    big = mb >= jnp.uint32(0x3FB504F3)
    mb = jnp.where(big, mb - jnp.uint32(0x00800000), mb)
    e = jnp.where(big, e + 1, e)
    m = lax.bitcast_convert_type(mb, jnp.float32)
    t = (m - jnp.float32(1.0)) / (m + jnp.float32(1.0))
    t2 = t * t
    p = jnp.float32(1.0 / 11.0)
    for cc in (1.0 / 9.0, 1.0 / 7.0, 1.0 / 5.0, 1.0 / 3.0, 1.0):
        p = p * t2 + jnp.float32(cc)
    return e.astype(jnp.float32) * jnp.float32(_LN2) + jnp.float32(2.0) * t * p


def _sc_score_chunk(buf, tvec, t0, row_global, r, j):
    iota = lax.broadcasted_iota(jnp.int32, (16,), 0)
    col = iota + (_V_TC + j * 16)
    cnt = (row_global * _V + col + _K1).astype(jnp.uint32)
    bits = _threefry_bits(cnt)
    fb = lax.bitcast_convert_type(
        (bits >> jnp.uint32(9)) | jnp.uint32(0x3F800000), jnp.float32)
    uni = fb - jnp.float32(1.0)
    expo = -_sc_ln(jnp.float32(1.0) - uni)
    noise = _sc_ln(expo + jnp.float32(1e-10))
    lg = buf[r, pl.ds(j * 16, 16)]
    s = lg / tvec - noise
    se = jnp.where(t0, lg, s)
    return se, lg, col


def _sc_body(logits_hbm, temps_hbm, outi_hbm, outlg_hbm, buf, tbuf, obi, oblg):
    wid = lax.axis_index("s") * 2 + lax.axis_index("c")
    r0 = wid * _SC_ROWS

    pltpu.sync_copy(logits_hbm.at[pl.ds(r0, _SC_ROWS), pl.ds(_V_TC, _V_SC)], buf)
    pltpu.sync_copy(temps_hbm.at[pl.ds(r0, _SC_ROWS)], tbuf)

    neg_inf = jnp.float32(-jnp.inf)
    for r in range(_SC_ROWS):
        row_global = r0 + r
        tvec = tbuf[r, :]
        t0 = tvec == jnp.float32(0.0)

        def body(i, carry):
            acc_v, acc_i, acc_lg = carry
            for cc in range(_SC_UNROLL):
                se, lg, col = _sc_score_chunk(
                    buf, tvec, t0, row_global, r, i * _SC_UNROLL + cc)
                upd = se > acc_v
                acc_v = jnp.where(upd, se, acc_v)
                acc_i = jnp.where(upd, col, acc_i)
                acc_lg = jnp.where(upd, lg, acc_lg)
            return acc_v, acc_i, acc_lg

        init = (jnp.full((16,), neg_inf, jnp.float32),
                jnp.zeros((16,), jnp.int32),
                jnp.zeros((16,), jnp.float32))
        acc_v, acc_i, acc_lg = lax.fori_loop(0, _SC_TRIPS, body, init)
        for cc in range(_SC_TAIL):
            se, lg, col = _sc_score_chunk(
                buf, tvec, t0, row_global, r, _SC_TRIPS * _SC_UNROLL + cc)
            upd = se > acc_v
            acc_v = jnp.where(upd, se, acc_v)
            acc_i = jnp.where(upd, col, acc_i)
            acc_lg = jnp.where(upd, lg, acc_lg)

        obi[r, :] = acc_i
        oblg[r, :] = acc_lg

    pltpu.sync_copy(obi, outi_hbm.at[pl.ds(r0, _SC_ROWS)])
    pltpu.sync_copy(oblg, outlg_hbm.at[pl.ds(r0, _SC_ROWS)])


def _sc_candidates(logits, temps16):
    mesh = plsc.VectorSubcoreMesh(core_axis_name="c", subcore_axis_name="s")
    fn = pl.kernel(
        _sc_body,
        mesh=mesh,
        out_type=[
            jax.ShapeDtypeStruct((_B, 16), jnp.int32),
            jax.ShapeDtypeStruct((_B, 16), jnp.float32),
        ],
        scratch_types=[
            pltpu.VMEM((_SC_ROWS, _V_SC), jnp.float32),
            pltpu.VMEM((_SC_ROWS, 16), jnp.float32),
            pltpu.VMEM((_SC_ROWS, 16), jnp.int32),
            pltpu.VMEM((_SC_ROWS, 16), jnp.float32),
        ],
    )
    return fn(logits, temps16)


def _pass2_body(val_ref, idx_ref, sci_ref, sclg_ref, temp_ref, out_ref):
    v = val_ref[...]
    i = idx_ref[...]
    big = jnp.int32(_V)
    m_tc = jnp.max(v, axis=1, keepdims=True)
    tok_tc = jnp.min(jnp.where(v == m_tc, i, big), axis=1, keepdims=True)

    temp = temp_ref[...]
    t0 = temp == jnp.float32(0.0)
    sci = sci_ref[...]
    sclg = sclg_ref[...]
    row = lax.broadcasted_iota(jnp.int32, (_B, 16), 0)
    cnt = (row * _V + sci + jnp.int32(_K1)).astype(jnp.uint32)
    noise = _bits_to_noise(_threefry_bits(cnt))
    s = sclg / temp - noise
    se = jnp.where(t0, sclg, s)
    m_sc = jnp.max(se, axis=1, keepdims=True)
    tok_sc = jnp.min(jnp.where(se == m_sc, sci, big), axis=1, keepdims=True)

    tok = jnp.where(m_sc > m_tc, tok_sc, tok_tc)
    out_ref[...] = jnp.broadcast_to(tok, (_B, _LANES))


def kernel(logits, temperatures):
    logits = logits.astype(jnp.float32)
    temps = temperatures.astype(jnp.float32)
    temps2d = temps[:, None]
    temps16 = jnp.broadcast_to(temps2d, (_B, 16))

    sci, sclg = _sc_candidates(logits, temps16)

    val, idx = pl.pallas_call(
        _pass1_body,
        grid=(_B // _ROWS, _NCB),
        in_specs=[
            pl.BlockSpec((_ROWS, _CW), lambda r, c: (r, c)),
            pl.BlockSpec((_ROWS, 1), lambda r, c: (r, 0)),
        ],
        out_specs=[
            pl.BlockSpec((_ROWS, _LANES), lambda r, c: (r, 0)),
            pl.BlockSpec((_ROWS, _LANES), lambda r, c: (r, 0)),
        ],
        out_shape=[
            jax.ShapeDtypeStruct((_B, _LANES), jnp.float32),
            jax.ShapeDtypeStruct((_B, _LANES), jnp.int32),
        ],
        compiler_params=pltpu.CompilerParams(
            dimension_semantics=("parallel", "arbitrary"),
        ),
    )(logits, temps2d)

    out = pl.pallas_call(
        _pass2_body,
        out_shape=jax.ShapeDtypeStruct((_B, _LANES), jnp.int32),
    )(val, idx, sci, sclg, temps2d)
    return out[:, 0]


# speedup vs baseline: 1.0259x; 1.0259x over previous
import functools

import jax
import jax.numpy as jnp
from jax import lax
from jax.experimental import pallas as pl
from jax.experimental.pallas import tpu as pltpu
from jax.experimental.pallas import tpu_sc as plsc

_B = 128
_V = 100000
_V_TC = 84992
_V_SC = _V - _V_TC
_ROWS = 8
_CW = 21248
_NCB = _V_TC // _CW
_LANES = 128

_NW = 32
_SC_ROWS = _B // _NW
_SC_VREGS = _V_SC // 16
_SC_UNROLL = 8
_SC_TRIPS = _SC_VREGS // _SC_UNROLL
_SC_TAIL = _SC_VREGS - _SC_TRIPS * _SC_UNROLL

_K0 = 0
_K1 = 42
_K2 = 0x1BD11BDA ^ _K0 ^ _K1
_ROT_A = (13, 15, 26, 6)
_ROT_B = (17, 29, 16, 24)

_LN2 = 0.69314718055994530942


def _threefry_bits(x1):
    u32 = lambda v: jnp.uint32(v)

    def four_rounds(x0, x1, rots):
        for d in rots:
            x0 = x0 + x1
            x1 = (x1 << u32(d)) | (x1 >> u32(32 - d))
            x1 = x1 ^ x0
        return x0, x1

    x0 = x1
    x1 = (x1 << u32(_ROT_A[0])) | (x1 >> u32(32 - _ROT_A[0]))
    x1 = x1 ^ x0
    for d in _ROT_A[1:]:
        x0 = x0 + x1
        x1 = (x1 << u32(d)) | (x1 >> u32(32 - d))
        x1 = x1 ^ x0

    x0 = x0 + u32(_K1); x1 = x1 + u32(_K2 + 1)
    x0, x1 = four_rounds(x0, x1, _ROT_B)
    x0 = x0 + u32(_K2); x1 = x1 + u32(_K0 + 2)
    x0, x1 = four_rounds(x0, x1, _ROT_A)
    x0 = x0 + u32(_K0); x1 = x1 + u32(_K1 + 3)
    x0, x1 = four_rounds(x0, x1, _ROT_B)
    x0 = x0 + u32(_K1); x1 = x1 + u32(_K2 + 4)
    x0, x1 = four_rounds(x0, x1, _ROT_A)
    x0 = x0 + u32(_K2); x1 = x1 + u32(_K0 + 5)
    return x0 ^ x1


def _bits_to_noise(bits):
    fb = lax.bitcast_convert_type(
        (bits >> jnp.uint32(9)) | jnp.uint32(0x3F800000), jnp.float32)
    uni = fb - jnp.float32(1.0)
    expo = -jnp.log1p(-uni)
    return jnp.log(expo + jnp.float32(1e-10))


def _pass1_body(logits_ref, temp_ref, val_ref, idx_ref):
    r = pl.program_id(0)
    c = pl.program_id(1)

    temp = jnp.broadcast_to(temp_ref[...], (_ROWS, _LANES))
    t0 = temp == jnp.float32(0.0)
    neg_inf = jnp.float32(-jnp.inf)

    lane = lax.broadcasted_iota(jnp.int32, (_ROWS, _LANES), 1)
    base_col = lane + c * _CW
    row = lax.broadcasted_iota(jnp.int32, (_ROWS, _LANES), 0) + r * _ROWS
    cnt_base = (row * _V + base_col + jnp.int32(_K1)).astype(jnp.uint32)

    first = c == 0
    acc_v = jnp.where(first, jnp.full((_ROWS, _LANES), neg_inf), val_ref[...])
    acc_i = jnp.where(first, jnp.zeros((_ROWS, _LANES), jnp.int32), idx_ref[...])

    for k in range(_CW // _LANES):
        col = base_col + jnp.int32(k * _LANES)
        bits = _threefry_bits(cnt_base + jnp.uint32(k * _LANES))
        noise = _bits_to_noise(bits)

        lg = logits_ref[:, k * _LANES:(k + 1) * _LANES]
        s = lg / temp - noise
        se = jnp.where(t0, lg, s)

        upd = se > acc_v
        acc_v = jnp.where(upd, se, acc_v)
        acc_i = jnp.where(upd, col, acc_i)

    val_ref[...] = acc_v
    idx_ref[...] = acc_i


def _sc_ln(x):
    xb = lax.bitcast_convert_type(x, jnp.uint32)
    e = (xb >> jnp.uint32(23)).astype(jnp.int32) - 127
    mb = (xb & jnp.uint32(0x7FFFFF)) | jnp.uint32(0x3F800000)
    big = mb >= jnp.uint32(0x3FB504F3)
    mb = jnp.where(big, mb - jnp.uint32(0x00800000), mb)
    e = jnp.where(big, e + 1, e)
    m = lax.bitcast_convert_type(mb, jnp.float32)
    t = (m - jnp.float32(1.0)) / (m + jnp.float32(1.0))
    t2 = t * t
    p = jnp.float32(1.0 / 11.0)
    for cc in (1.0 / 9.0, 1.0 / 7.0, 1.0 / 5.0, 1.0 / 3.0, 1.0):
        p = p * t2 + jnp.float32(cc)
    return e.astype(jnp.float32) * jnp.float32(_LN2) + jnp.float32(2.0) * t * p


def _sc_score_chunk(buf, tvec, t0, row_global, r, j):
    iota = lax.broadcasted_iota(jnp.int32, (16,), 0)
    col = iota + (_V_TC + j * 16)
    cnt = (row_global * _V + col + _K1).astype(jnp.uint32)
    bits = _threefry_bits(cnt)
    fb = lax.bitcast_convert_type(
        (bits >> jnp.uint32(9)) | jnp.uint32(0x3F800000), jnp.float32)
    uni = fb - jnp.float32(1.0)
    expo = -_sc_ln(jnp.float32(1.0) - uni)
    noise = _sc_ln(expo + jnp.float32(1e-10))
    lg = buf[r, pl.ds(j * 16, 16)]
    s = lg / tvec - noise
    se = jnp.where(t0, lg, s)
    return se, lg, col


def _sc_body(logits_hbm, temps_hbm, outi_hbm, outlg_hbm, buf, tbuf, obi, oblg):
    wid = lax.axis_index("s") * 2 + lax.axis_index("c")
    r0 = wid * _SC_ROWS

    pltpu.sync_copy(logits_hbm.at[pl.ds(r0, _SC_ROWS), pl.ds(_V_TC, _V_SC)], buf)
    pltpu.sync_copy(temps_hbm.at[pl.ds(r0, _SC_ROWS)], tbuf)

    neg_inf = jnp.float32(-jnp.inf)
    for r in range(_SC_ROWS):
        row_global = r0 + r
        tvec = tbuf[r, :]
        t0 = tvec == jnp.float32(0.0)

        def body(i, carry):
            acc_v, acc_i, acc_lg = carry
            for cc in range(_SC_UNROLL):
                se, lg, col = _sc_score_chunk(
                    buf, tvec, t0, row_global, r, i * _SC_UNROLL + cc)
                upd = se > acc_v
                acc_v = jnp.where(upd, se, acc_v)
                acc_i = jnp.where(upd, col, acc_i)
                acc_lg = jnp.where(upd, lg, acc_lg)
            return acc_v, acc_i, acc_lg

        init = (jnp.full((16,), neg_inf, jnp.float32),
                jnp.zeros((16,), jnp.int32),
                jnp.zeros((16,), jnp.float32))
        acc_v, acc_i, acc_lg = lax.fori_loop(0, _SC_TRIPS, body, init)
        for cc in range(_SC_TAIL):
            se, lg, col = _sc_score_chunk(
                buf, tvec, t0, row_global, r, _SC_TRIPS * _SC_UNROLL + cc)
            upd = se > acc_v
            acc_v = jnp.where(upd, se, acc_v)
            acc_i = jnp.where(upd, col, acc_i)
            acc_lg = jnp.where(upd, lg, acc_lg)

        obi[r, :] = acc_i
        oblg[r, :] = acc_lg

    pltpu.sync_copy(obi, outi_hbm.at[pl.ds(r0, _SC_ROWS)])
    pltpu.sync_copy(oblg, outlg_hbm.at[pl.ds(r0, _SC_ROWS)])


def _sc_candidates(logits, temps16):
    mesh = plsc.VectorSubcoreMesh(core_axis_name="c", subcore_axis_name="s")
    fn = pl.kernel(
        _sc_body,
        mesh=mesh,
        out_type=[
            jax.ShapeDtypeStruct((_B, 16), jnp.int32),
            jax.ShapeDtypeStruct((_B, 16), jnp.float32),
        ],
        scratch_types=[
            pltpu.VMEM((_SC_ROWS, _V_SC), jnp.float32),
            pltpu.VMEM((_SC_ROWS, 16), jnp.float32),
            pltpu.VMEM((_SC_ROWS, 16), jnp.int32),
            pltpu.VMEM((_SC_ROWS, 16), jnp.float32),
        ],
    )
    return fn(logits, temps16)


def _pass2_body(val_ref, idx_ref, sci_ref, sclg_ref, temp_ref, out_ref):
    v = val_ref[...]
    i = idx_ref[...]
    big = jnp.int32(_V)
    m_tc = jnp.max(v, axis=1, keepdims=True)
    tok_tc = jnp.min(jnp.where(v == m_tc, i, big), axis=1, keepdims=True)

    temp = temp_ref[...]
    t0 = temp == jnp.float32(0.0)
    sci = sci_ref[...]
    sclg = sclg_ref[...]
    row = lax.broadcasted_iota(jnp.int32, (_B, 16), 0)
    cnt = (row * _V + sci + jnp.int32(_K1)).astype(jnp.uint32)
    noise = _bits_to_noise(_threefry_bits(cnt))
    s = sclg / temp - noise
    se = jnp.where(t0, sclg, s)
    m_sc = jnp.max(se, axis=1, keepdims=True)
    tok_sc = jnp.min(jnp.where(se == m_sc, sci, big), axis=1, keepdims=True)

    tok = jnp.where(m_sc > m_tc, tok_sc, tok_tc)
    out_ref[...] = jnp.broadcast_to(tok, (_B, _LANES))


def kernel(logits, temperatures):
    logits = logits.astype(jnp.float32)
    temps = temperatures.astype(jnp.float32)
    temps2d = temps[:, None]
    temps16 = jnp.broadcast_to(temps2d, (_B, 16))

    sci, sclg = _sc_candidates(logits, temps16)

    val, idx = pl.pallas_call(
        _pass1_body,
        grid=(_B // _ROWS, _NCB),
        in_specs=[
            pl.BlockSpec((_ROWS, _CW), lambda r, c: (r, c)),
            pl.BlockSpec((_ROWS, 1), lambda r, c: (r, 0)),
        ],
        out_specs=[
            pl.BlockSpec((_ROWS, _LANES), lambda r, c: (r, 0)),
            pl.BlockSpec((_ROWS, _LANES), lambda r, c: (r, 0)),
        ],
        out_shape=[
            jax.ShapeDtypeStruct((_B, _LANES), jnp.float32),
            jax.ShapeDtypeStruct((_B, _LANES), jnp.int32),
        ],
        compiler_params=pltpu.CompilerParams(
            dimension_semantics=("parallel", "arbitrary"),
        ),
    )(logits, temps2d)

    out = pl.pallas_call(
        _pass2_body,
        out_shape=jax.ShapeDtypeStruct((_B, _LANES), jnp.int32),
    )(val, idx, sci, sclg, temps2d)
    return out[:, 0]


# speedup vs baseline: 1.0465x; 1.0200x over previous
import functools

import jax
import jax.numpy as jnp
from jax import lax
from jax.experimental import pallas as pl
from jax.experimental.pallas import tpu as pltpu
from jax.experimental.pallas import tpu_sc as plsc

_B = 128
_V = 100000
_V_TC = 83456
_V_SC = _V - _V_TC
_ROWS = 8
_CW = 20864
_NCB = _V_TC // _CW
_LANES = 128

_NW = 32
_SC_ROWS = _B // _NW
_SC_VREGS = _V_SC // 16
_SC_UNROLL = 8
_SC_TRIPS = _SC_VREGS // _SC_UNROLL
_SC_TAIL = _SC_VREGS - _SC_TRIPS * _SC_UNROLL

_K0 = 0
_K1 = 42
_K2 = 0x1BD11BDA ^ _K0 ^ _K1
_ROT_A = (13, 15, 26, 6)
_ROT_B = (17, 29, 16, 24)

_LN2 = 0.69314718055994530942


def _threefry_bits(x1):
    u32 = lambda v: jnp.uint32(v)

    def four_rounds(x0, x1, rots):
        for d in rots:
            x0 = x0 + x1
            x1 = (x1 << u32(d)) | (x1 >> u32(32 - d))
            x1 = x1 ^ x0
        return x0, x1

    x0 = x1
    x1 = (x1 << u32(_ROT_A[0])) | (x1 >> u32(32 - _ROT_A[0]))
    x1 = x1 ^ x0
    for d in _ROT_A[1:]:
        x0 = x0 + x1
        x1 = (x1 << u32(d)) | (x1 >> u32(32 - d))
        x1 = x1 ^ x0

    x0 = x0 + u32(_K1); x1 = x1 + u32(_K2 + 1)
    x0, x1 = four_rounds(x0, x1, _ROT_B)
    x0 = x0 + u32(_K2); x1 = x1 + u32(_K0 + 2)
    x0, x1 = four_rounds(x0, x1, _ROT_A)
    x0 = x0 + u32(_K0); x1 = x1 + u32(_K1 + 3)
    x0, x1 = four_rounds(x0, x1, _ROT_B)
    x0 = x0 + u32(_K1); x1 = x1 + u32(_K2 + 4)
    x0, x1 = four_rounds(x0, x1, _ROT_A)
    x0 = x0 + u32(_K2); x1 = x1 + u32(_K0 + 5)
    return x0 ^ x1


def _bits_to_noise(bits):
    fb = lax.bitcast_convert_type(
        (bits >> jnp.uint32(9)) | jnp.uint32(0x3F800000), jnp.float32)
    uni = fb - jnp.float32(1.0)
    expo = -jnp.log1p(-uni)
    return jnp.log(expo + jnp.float32(1e-10))


def _pass1_body(logits_ref, temp_ref, val_ref, idx_ref):
    r = pl.program_id(0)
    c = pl.program_id(1)

    temp = jnp.broadcast_to(temp_ref[...], (_ROWS, _LANES))
    t0 = temp == jnp.float32(0.0)
    neg_inf = jnp.float32(-jnp.inf)

    lane = lax.broadcasted_iota(jnp.int32, (_ROWS, _LANES), 1)
    base_col = lane + c * _CW
    row = lax.broadcasted_iota(jnp.int32, (_ROWS, _LANES), 0) + r * _ROWS
    cnt_base = (row * _V + base_col + jnp.int32(_K1)).astype(jnp.uint32)

    first = c == 0
    acc_v = jnp.where(first, jnp.full((_ROWS, _LANES), neg_inf), val_ref[...])
    acc_i = jnp.where(first, jnp.zeros((_ROWS, _LANES), jnp.int32), idx_ref[...])

    for k in range(_CW // _LANES):
        col = base_col + jnp.int32(k * _LANES)
        bits = _threefry_bits(cnt_base + jnp.uint32(k * _LANES))
        noise = _bits_to_noise(bits)

        lg = logits_ref[:, k * _LANES:(k + 1) * _LANES]
        s = lg / temp - noise
        se = jnp.where(t0, lg, s)

        upd = se > acc_v
        acc_v = jnp.where(upd, se, acc_v)
        acc_i = jnp.where(upd, col, acc_i)

    val_ref[...] = acc_v
    idx_ref[...] = acc_i


def _sc_ln(x):
    xb = lax.bitcast_convert_type(x, jnp.uint32)
    e = (xb >> jnp.uint32(23)).astype(jnp.int32) - 127
    mb = (xb & jnp.uint32(0x7FFFFF)) | jnp.uint32(0x3F800000)
    big = mb >= jnp.uint32(0x3FB504F3)
    mb = jnp.where(big, mb - jnp.uint32(0x00800000), mb)
    e = jnp.where(big, e + 1, e)
    m = lax.bitcast_convert_type(mb, jnp.float32)
    t = (m - jnp.float32(1.0)) / (m + jnp.float32(1.0))
    t2 = t * t
    p = jnp.float32(1.0 / 11.0)
    for cc in (1.0 / 9.0, 1.0 / 7.0, 1.0 / 5.0, 1.0 / 3.0, 1.0):
        p = p * t2 + jnp.float32(cc)
    return e.astype(jnp.float32) * jnp.float32(_LN2) + jnp.float32(2.0) * t * p


def _sc_score_chunk(buf, tvec, t0, row_global, r, j):
    iota = lax.broadcasted_iota(jnp.int32, (16,), 0)
    col = iota + (_V_TC + j * 16)
    cnt = (row_global * _V + col + _K1).astype(jnp.uint32)
    bits = _threefry_bits(cnt)
    fb = lax.bitcast_convert_type(
        (bits >> jnp.uint32(9)) | jnp.uint32(0x3F800000), jnp.float32)
    uni = fb - jnp.float32(1.0)
    expo = -_sc_ln(jnp.float32(1.0) - uni)
    noise = _sc_ln(expo + jnp.float32(1e-10))
    lg = buf[r, pl.ds(j * 16, 16)]
    s = lg / tvec - noise
    se = jnp.where(t0, lg, s)
    return se, lg, col


def _sc_body(logits_hbm, temps_hbm, outi_hbm, outlg_hbm, buf, tbuf, obi, oblg):
    wid = lax.axis_index("s") * 2 + lax.axis_index("c")
    r0 = wid * _SC_ROWS

    pltpu.sync_copy(logits_hbm.at[pl.ds(r0, _SC_ROWS), pl.ds(_V_TC, _V_SC)], buf)
    pltpu.sync_copy(temps_hbm.at[pl.ds(r0, _SC_ROWS)], tbuf)

    neg_inf = jnp.float32(-jnp.inf)
    for r in range(_SC_ROWS):
        row_global = r0 + r
        tvec = tbuf[r, :]
        t0 = tvec == jnp.float32(0.0)

        def body(i, carry):
            acc_v, acc_i, acc_lg = carry
            for cc in range(_SC_UNROLL):
                se, lg, col = _sc_score_chunk(
                    buf, tvec, t0, row_global, r, i * _SC_UNROLL + cc)
                upd = se > acc_v
                acc_v = jnp.where(upd, se, acc_v)
                acc_i = jnp.where(upd, col, acc_i)
                acc_lg = jnp.where(upd, lg, acc_lg)
            return acc_v, acc_i, acc_lg

        init = (jnp.full((16,), neg_inf, jnp.float32),
                jnp.zeros((16,), jnp.int32),
                jnp.zeros((16,), jnp.float32))
        acc_v, acc_i, acc_lg = lax.fori_loop(0, _SC_TRIPS, body, init)
        for cc in range(_SC_TAIL):
            se, lg, col = _sc_score_chunk(
                buf, tvec, t0, row_global, r, _SC_TRIPS * _SC_UNROLL + cc)
            upd = se > acc_v
            acc_v = jnp.where(upd, se, acc_v)
            acc_i = jnp.where(upd, col, acc_i)
            acc_lg = jnp.where(upd, lg, acc_lg)

        obi[r, :] = acc_i
        oblg[r, :] = acc_lg

    pltpu.sync_copy(obi, outi_hbm.at[pl.ds(r0, _SC_ROWS)])
    pltpu.sync_copy(oblg, outlg_hbm.at[pl.ds(r0, _SC_ROWS)])


def _sc_candidates(logits, temps16):
    mesh = plsc.VectorSubcoreMesh(core_axis_name="c", subcore_axis_name="s")
    fn = pl.kernel(
        _sc_body,
        mesh=mesh,
        out_type=[
            jax.ShapeDtypeStruct((_B, 16), jnp.int32),
            jax.ShapeDtypeStruct((_B, 16), jnp.float32),
        ],
        scratch_types=[
            pltpu.VMEM((_SC_ROWS, _V_SC), jnp.float32),
            pltpu.VMEM((_SC_ROWS, 16), jnp.float32),
            pltpu.VMEM((_SC_ROWS, 16), jnp.int32),
            pltpu.VMEM((_SC_ROWS, 16), jnp.float32),
        ],
    )
    return fn(logits, temps16)


def _pass2_body(val_ref, idx_ref, sci_ref, sclg_ref, temp_ref, out_ref):
    v = val_ref[...]
    i = idx_ref[...]
    big = jnp.int32(_V)
    m_tc = jnp.max(v, axis=1, keepdims=True)
    tok_tc = jnp.min(jnp.where(v == m_tc, i, big), axis=1, keepdims=True)

    temp = temp_ref[...]
    t0 = temp == jnp.float32(0.0)
    sci = sci_ref[...]
    sclg = sclg_ref[...]
    row = lax.broadcasted_iota(jnp.int32, (_B, 16), 0)
    cnt = (row * _V + sci + jnp.int32(_K1)).astype(jnp.uint32)
    noise = _bits_to_noise(_threefry_bits(cnt))
    s = sclg / temp - noise
    se = jnp.where(t0, sclg, s)
    m_sc = jnp.max(se, axis=1, keepdims=True)
    tok_sc = jnp.min(jnp.where(se == m_sc, sci, big), axis=1, keepdims=True)

    tok = jnp.where(m_sc > m_tc, tok_sc, tok_tc)
    out_ref[...] = jnp.broadcast_to(tok, (_B, _LANES))


def kernel(logits, temperatures):
    logits = logits.astype(jnp.float32)
    temps = temperatures.astype(jnp.float32)
    temps2d = temps[:, None]
    temps16 = jnp.broadcast_to(temps2d, (_B, 16))

    sci, sclg = _sc_candidates(logits, temps16)

    val, idx = pl.pallas_call(
        _pass1_body,
        grid=(_B // _ROWS, _NCB),
        in_specs=[
            pl.BlockSpec((_ROWS, _CW), lambda r, c: (r, c)),
            pl.BlockSpec((_ROWS, 1), lambda r, c: (r, 0)),
        ],
        out_specs=[
            pl.BlockSpec((_ROWS, _LANES), lambda r, c: (r, 0)),
            pl.BlockSpec((_ROWS, _LANES), lambda r, c: (r, 0)),
        ],
        out_shape=[
            jax.ShapeDtypeStruct((_B, _LANES), jnp.float32),
            jax.ShapeDtypeStruct((_B, _LANES), jnp.int32),
        ],
        compiler_params=pltpu.CompilerParams(
            dimension_semantics=("parallel", "arbitrary"),
        ),
    )(logits, temps2d)

    out = pl.pallas_call(
        _pass2_body,
        out_shape=jax.ShapeDtypeStruct((_B, _LANES), jnp.int32),
    )(val, idx, sci, sclg, temps2d)
    return out[:, 0]
